# baseline (speedup 1.0000x reference)
import functools

import jax
import jax.numpy as jnp
from jax import lax
from jax.experimental import pallas as pl
from jax.experimental.pallas import tpu as pltpu
from jax.experimental.pallas import tpu_sc as plsc

DIM = 32
EPS = 1e-08
NW = 32
BB = 128
NBUF = 3


def _rsqrt16(x):
    i = lax.bitcast_convert_type(x, jnp.int32)
    y = lax.bitcast_convert_type(
        jnp.int32(0x5F3759DF) - lax.shift_right_logical(i, 1), jnp.float32
    )
    for _ in range(3):
        y = y * (1.5 - 0.5 * x * y * y)
    return jnp.where(x < jnp.float32(EPS * EPS), jnp.float32(1.0 / EPS), y)


def _make_kernel(n_seq, n_batch):
    mesh = plsc.VectorSubcoreMesh(core_axis_name="c", subcore_axis_name="s")

    @functools.partial(
        pl.kernel,
        out_type=(
            jax.ShapeDtypeStruct((n_seq, DIM, n_batch), jnp.float32),
            jax.ShapeDtypeStruct((n_seq, DIM, n_batch), jnp.float32),
            jax.ShapeDtypeStruct((n_seq, 2 * DIM, n_batch), jnp.float32),
        ),
        mesh=mesh,
        scratch_types=[
            pltpu.VMEM((NBUF, BB), jnp.int32),
            pltpu.VMEM((NBUF, BB, DIM), jnp.float32),
            pltpu.VMEM((NBUF, BB, DIM), jnp.float32),
            pltpu.VMEM((NBUF, DIM, BB), jnp.float32),
            pltpu.VMEM((NBUF, DIM, BB), jnp.float32),
            pltpu.VMEM((NBUF, 2 * DIM, BB), jnp.float32),
            pltpu.SemaphoreType.DMA((NBUF,)),
            pltpu.SemaphoreType.DMA((NBUF,)),
        ],
        compiler_params=pltpu.CompilerParams(
            needs_layout_passes=False, use_tc_tiling_on_sc=False
        ),
    )
    def kern(ids_hbm, base_hbm, ctx_hbm, qb_hbm, qc_hbm, qt_hbm,
             idx_v, base_v, ctx_v, tb_v, tc_v, tt_v, gsem, osem):
        wid = lax.axis_index("s") * 2 + lax.axis_index("c")
        col0 = wid * BB
        lanes = lax.iota(jnp.int32, 16)

        def issue_gathers(s, slot):
            pltpu.sync_copy(ids_hbm.at[s, wid], idx_v.at[slot])
            pltpu.async_copy(
                base_hbm.at[idx_v.at[slot]], base_v.at[slot], gsem.at[slot])
            pltpu.async_copy(
                ctx_hbm.at[idx_v.at[slot]], ctx_v.at[slot], gsem.at[slot])

        def wait_gathers(s, slot):
            pltpu.make_async_copy(
                base_hbm.at[idx_v.at[slot]], base_v.at[slot],
                gsem.at[slot]).wait()
            pltpu.make_async_copy(
                ctx_hbm.at[idx_v.at[slot]], ctx_v.at[slot],
                gsem.at[slot]).wait()

        def issue_outputs(s, slot):
            dst = pl.ds(col0, BB)
            pltpu.async_copy(
                tb_v.at[slot], qb_hbm.at[s, :, dst], osem.at[slot])
            pltpu.async_copy(
                tc_v.at[slot], qc_hbm.at[s, :, dst], osem.at[slot])
            pltpu.async_copy(
                tt_v.at[slot], qt_hbm.at[s, :, dst], osem.at[slot])

        def drain_outputs(s, slot):
            dst = pl.ds(col0, BB)
            pltpu.make_async_copy(
                tb_v.at[slot], qb_hbm.at[s, :, dst], osem.at[slot]).wait()
            pltpu.make_async_copy(
                tc_v.at[slot], qc_hbm.at[s, :, dst], osem.at[slot]).wait()
            pltpu.make_async_copy(
                tt_v.at[slot], qt_hbm.at[s, :, dst], osem.at[slot]).wait()

        def compute(slot):
            def group_body(g, _):
                r0 = g * 16
                ridx = r0 + lanes
                ssq = jnp.zeros((16,), jnp.float32)
                for d in range(DIM):
                    didx = jnp.full((16,), d, jnp.int32)
                    vb = plsc.load_gather(base_v.at[slot], [ridx, didx])
                    tb_v[slot, d, pl.ds(r0, 16)] = vb
                    ssq = ssq + vb * vb
                for d in range(DIM):
                    didx = jnp.full((16,), d, jnp.int32)
                    vc = plsc.load_gather(ctx_v.at[slot], [ridx, didx])
                    tc_v[slot, d, pl.ds(r0, 16)] = vc
                    ssq = ssq + vc * vc
                inv = _rsqrt16(ssq)
                for d in range(DIM):
                    tt_v[slot, d, pl.ds(r0, 16)] = (
                        tb_v[slot, d, pl.ds(r0, 16)] * inv)
                    tt_v[slot, DIM + d, pl.ds(r0, 16)] = (
                        tc_v[slot, d, pl.ds(r0, 16)] * inv)
                return 0

            lax.fori_loop(0, BB // 16, group_body, 0)

        issue_gathers(0, 0)

        def tile_body(s, _):
            slot = lax.rem(s, NBUF)
            nslot = lax.rem(s + 1, NBUF)

            @pl.when(s >= 2)
            def _():
                drain_outputs(s - 2, nslot)

            @pl.when(s + 1 < n_seq)
            def _():
                issue_gathers(s + 1, nslot)

            wait_gathers(s, slot)
            compute(slot)
            issue_outputs(s, slot)
            return 0

        lax.fori_loop(0, n_seq, tile_body, 0)
        drain_outputs(n_seq - 2, lax.rem(n_seq - 2, NBUF))
        drain_outputs(n_seq - 1, lax.rem(n_seq - 1, NBUF))

    return kern


def kernel(concept_ids, base_table, context_table):
    b, s = concept_ids.shape
    ids_t = concept_ids.T.reshape(s, NW, BB).astype(jnp.int32)
    qb_t, qc_t, qt_t = _make_kernel(s, b)(ids_t, base_table, context_table)
    return (
        qb_t.transpose(2, 0, 1),
        qc_t.transpose(2, 0, 1),
        qt_t.transpose(2, 0, 1),
    )


# speedup vs baseline: 1.0009x; 1.0009x over previous
import functools

import jax
import jax.numpy as jnp
from jax import lax
from jax.experimental import pallas as pl
from jax.experimental.pallas import tpu as pltpu
from jax.experimental.pallas import tpu_sc as plsc

DIM = 32
EPS = 1e-08
NW = 32
BB = 128
NBUF = 3


def _rsqrt16(x):
    i = lax.bitcast_convert_type(x, jnp.int32)
    y = lax.bitcast_convert_type(
        jnp.int32(0x5F3759DF) - lax.shift_right_logical(i, 1), jnp.float32
    )
    for _ in range(2):
        y = y * (1.5 - 0.5 * x * y * y)
    return jnp.where(x < jnp.float32(EPS * EPS), jnp.float32(1.0 / EPS), y)


def _make_kernel(n_seq, n_batch):
    mesh = plsc.VectorSubcoreMesh(core_axis_name="c", subcore_axis_name="s")

    @functools.partial(
        pl.kernel,
        out_type=(
            jax.ShapeDtypeStruct((n_seq, DIM, n_batch), jnp.float32),
            jax.ShapeDtypeStruct((n_seq, DIM, n_batch), jnp.float32),
            jax.ShapeDtypeStruct((n_seq, 2 * DIM, n_batch), jnp.float32),
        ),
        mesh=mesh,
        scratch_types=[
            pltpu.VMEM((NBUF, BB), jnp.int32),
            pltpu.VMEM((NBUF, BB, DIM), jnp.float32),
            pltpu.VMEM((NBUF, BB, DIM), jnp.float32),
            pltpu.VMEM((NBUF, DIM, BB), jnp.float32),
            pltpu.VMEM((NBUF, DIM, BB), jnp.float32),
            pltpu.VMEM((NBUF, 2 * DIM, BB), jnp.float32),
            pltpu.SemaphoreType.DMA((NBUF,)),
            pltpu.SemaphoreType.DMA((NBUF,)),
        ],
        compiler_params=pltpu.CompilerParams(
            needs_layout_passes=False, use_tc_tiling_on_sc=False
        ),
    )
    def kern(ids_hbm, base_hbm, ctx_hbm, qb_hbm, qc_hbm, qt_hbm,
             idx_v, base_v, ctx_v, tb_v, tc_v, tt_v, gsem, osem):
        wid = lax.axis_index("s") * 2 + lax.axis_index("c")
        col0 = wid * BB
        lanes = lax.iota(jnp.int32, 16)

        def issue_gathers(s, slot):
            pltpu.sync_copy(ids_hbm.at[s, wid], idx_v.at[slot])
            pltpu.async_copy(
                base_hbm.at[idx_v.at[slot]], base_v.at[slot], gsem.at[slot])
            pltpu.async_copy(
                ctx_hbm.at[idx_v.at[slot]], ctx_v.at[slot], gsem.at[slot])

        def wait_gathers(s, slot):
            pltpu.make_async_copy(
                base_hbm.at[idx_v.at[slot]], base_v.at[slot],
                gsem.at[slot]).wait()
            pltpu.make_async_copy(
                ctx_hbm.at[idx_v.at[slot]], ctx_v.at[slot],
                gsem.at[slot]).wait()

        def issue_outputs(s, slot):
            dst = pl.ds(col0, BB)
            pltpu.async_copy(
                tb_v.at[slot], qb_hbm.at[s, :, dst], osem.at[slot])
            pltpu.async_copy(
                tc_v.at[slot], qc_hbm.at[s, :, dst], osem.at[slot])
            pltpu.async_copy(
                tt_v.at[slot], qt_hbm.at[s, :, dst], osem.at[slot])

        def drain_outputs(s, slot):
            dst = pl.ds(col0, BB)
            pltpu.make_async_copy(
                tb_v.at[slot], qb_hbm.at[s, :, dst], osem.at[slot]).wait()
            pltpu.make_async_copy(
                tc_v.at[slot], qc_hbm.at[s, :, dst], osem.at[slot]).wait()
            pltpu.make_async_copy(
                tt_v.at[slot], qt_hbm.at[s, :, dst], osem.at[slot]).wait()

        def compute(slot):
            def group_body(g, _):
                r0 = g * 16
                ridx = r0 + lanes
                acc = [jnp.zeros((16,), jnp.float32) for _ in range(4)]
                for d in range(DIM):
                    didx = jnp.full((16,), d, jnp.int32)
                    vb = plsc.load_gather(base_v.at[slot], [ridx, didx])
                    tb_v[slot, d, pl.ds(r0, 16)] = vb
                    acc[d % 4] = acc[d % 4] + vb * vb
                for d in range(DIM):
                    didx = jnp.full((16,), d, jnp.int32)
                    vc = plsc.load_gather(ctx_v.at[slot], [ridx, didx])
                    tc_v[slot, d, pl.ds(r0, 16)] = vc
                    acc[d % 4] = acc[d % 4] + vc * vc
                inv = _rsqrt16((acc[0] + acc[1]) + (acc[2] + acc[3]))
                for d in range(DIM):
                    tt_v[slot, d, pl.ds(r0, 16)] = (
                        tb_v[slot, d, pl.ds(r0, 16)] * inv)
                    tt_v[slot, DIM + d, pl.ds(r0, 16)] = (
                        tc_v[slot, d, pl.ds(r0, 16)] * inv)
                return 0

            lax.fori_loop(0, BB // 16, group_body, 0)

        issue_gathers(0, 0)

        def tile_body(s, _):
            slot = lax.rem(s, NBUF)
            nslot = lax.rem(s + 1, NBUF)

            @pl.when(s >= 2)
            def _():
                drain_outputs(s - 2, nslot)

            @pl.when(s + 1 < n_seq)
            def _():
                issue_gathers(s + 1, nslot)

            wait_gathers(s, slot)
            compute(slot)
            issue_outputs(s, slot)
            return 0

        lax.fori_loop(0, n_seq, tile_body, 0)
        drain_outputs(n_seq - 2, lax.rem(n_seq - 2, NBUF))
        drain_outputs(n_seq - 1, lax.rem(n_seq - 1, NBUF))

    return kern


def kernel(concept_ids, base_table, context_table):
    b, s = concept_ids.shape
    ids_t = concept_ids.T.reshape(s, NW, BB).astype(jnp.int32)
    qb_t, qc_t, qt_t = _make_kernel(s, b)(ids_t, base_table, context_table)
    return (
        qb_t.transpose(2, 0, 1),
        qc_t.transpose(2, 0, 1),
        qt_t.transpose(2, 0, 1),
    )


# speedup vs baseline: 1.3447x; 1.3435x over previous
import functools

import jax
import jax.numpy as jnp
from jax import lax
from jax.experimental import pallas as pl
from jax.experimental.pallas import tpu as pltpu
from jax.experimental.pallas import tpu_sc as plsc

DIM = 32
EPS = 1e-08
NW = 32
BB = 128
NBUF = 3


def _rsqrt16(x):
    i = lax.bitcast_convert_type(x, jnp.int32)
    y = lax.bitcast_convert_type(
        jnp.int32(0x5F3759DF) - lax.shift_right_logical(i, 1), jnp.float32
    )
    for _ in range(2):
        y = y * (1.5 - 0.5 * x * y * y)
    return jnp.where(x < jnp.float32(EPS * EPS), jnp.float32(1.0 / EPS), y)


def _make_kernel(n_seq, n_batch):
    mesh = plsc.VectorSubcoreMesh(core_axis_name="c", subcore_axis_name="s")

    @functools.partial(
        pl.kernel,
        out_type=(
            jax.ShapeDtypeStruct((n_seq, DIM, n_batch), jnp.float32),
            jax.ShapeDtypeStruct((n_seq, DIM, n_batch), jnp.float32),
            jax.ShapeDtypeStruct((n_seq, 2 * DIM, n_batch), jnp.float32),
        ),
        mesh=mesh,
        scratch_types=[
            pltpu.VMEM((NBUF, BB), jnp.int32),
            pltpu.VMEM((NBUF, BB, DIM), jnp.float32),
            pltpu.VMEM((NBUF, BB, DIM), jnp.float32),
            pltpu.VMEM((NBUF, DIM, BB), jnp.float32),
            pltpu.VMEM((NBUF, DIM, BB), jnp.float32),
            pltpu.VMEM((NBUF, 2 * DIM, BB), jnp.float32),
            pltpu.SemaphoreType.DMA((NBUF,)),
            pltpu.SemaphoreType.DMA((NBUF,)),
        ],
        compiler_params=pltpu.CompilerParams(
            needs_layout_passes=False, use_tc_tiling_on_sc=False
        ),
    )
    def kern(ids_hbm, base_hbm, ctx_hbm, qb_hbm, qc_hbm, qt_hbm,
             idx_v, base_v, ctx_v, tb_v, tc_v, tt_v, gsem, osem):
        wid = lax.axis_index("s") * 2 + lax.axis_index("c")
        col0 = wid * BB
        lanes = lax.iota(jnp.int32, 16)

        def issue_gathers(s, slot):
            pltpu.sync_copy(ids_hbm.at[s, wid], idx_v.at[slot])
            pltpu.async_copy(
                base_hbm.at[idx_v.at[slot]], base_v.at[slot], gsem.at[slot])
            pltpu.async_copy(
                ctx_hbm.at[idx_v.at[slot]], ctx_v.at[slot], gsem.at[slot])

        def wait_gathers(s, slot):
            pltpu.make_async_copy(
                base_hbm.at[idx_v.at[slot]], base_v.at[slot],
                gsem.at[slot]).wait()
            pltpu.make_async_copy(
                ctx_hbm.at[idx_v.at[slot]], ctx_v.at[slot],
                gsem.at[slot]).wait()

        def issue_outputs(s, slot):
            dst = pl.ds(col0, BB)
            pltpu.async_copy(
                tb_v.at[slot], qb_hbm.at[s, :, dst], osem.at[slot])
            pltpu.async_copy(
                tc_v.at[slot], qc_hbm.at[s, :, dst], osem.at[slot])
            pltpu.async_copy(
                tt_v.at[slot], qt_hbm.at[s, :, dst], osem.at[slot])

        def drain_outputs(s, slot):
            dst = pl.ds(col0, BB)
            pltpu.make_async_copy(
                tb_v.at[slot], qb_hbm.at[s, :, dst], osem.at[slot]).wait()
            pltpu.make_async_copy(
                tc_v.at[slot], qc_hbm.at[s, :, dst], osem.at[slot]).wait()
            pltpu.make_async_copy(
                tt_v.at[slot], qt_hbm.at[s, :, dst], osem.at[slot]).wait()

        def compute(slot):
            def group_body(g, _):
                r0 = g * 16
                ridx = r0 + lanes
                acc = [jnp.zeros((16,), jnp.float32) for _ in range(4)]
                for d in range(DIM):
                    didx = (lanes + d) & (DIM - 1)
                    vb = plsc.load_gather(base_v.at[slot], [ridx, didx])
                    plsc.store_scatter(tb_v.at[slot], [didx, ridx], vb)
                    acc[d % 4] = acc[d % 4] + vb * vb
                for d in range(DIM):
                    didx = (lanes + d) & (DIM - 1)
                    vc = plsc.load_gather(ctx_v.at[slot], [ridx, didx])
                    plsc.store_scatter(tc_v.at[slot], [didx, ridx], vc)
                    acc[d % 4] = acc[d % 4] + vc * vc
                inv = _rsqrt16((acc[0] + acc[1]) + (acc[2] + acc[3]))
                for d in range(DIM):
                    tt_v[slot, d, pl.ds(r0, 16)] = (
                        tb_v[slot, d, pl.ds(r0, 16)] * inv)
                    tt_v[slot, DIM + d, pl.ds(r0, 16)] = (
                        tc_v[slot, d, pl.ds(r0, 16)] * inv)
                return 0

            lax.fori_loop(0, BB // 16, group_body, 0)

        issue_gathers(0, 0)

        def tile_body(s, _):
            slot = lax.rem(s, NBUF)
            nslot = lax.rem(s + 1, NBUF)

            @pl.when(s >= 2)
            def _():
                drain_outputs(s - 2, nslot)

            @pl.when(s + 1 < n_seq)
            def _():
                issue_gathers(s + 1, nslot)

            wait_gathers(s, slot)
            compute(slot)
            issue_outputs(s, slot)
            return 0

        lax.fori_loop(0, n_seq, tile_body, 0)
        drain_outputs(n_seq - 2, lax.rem(n_seq - 2, NBUF))
        drain_outputs(n_seq - 1, lax.rem(n_seq - 1, NBUF))

    return kern


def kernel(concept_ids, base_table, context_table):
    b, s = concept_ids.shape
    ids_t = concept_ids.T.reshape(s, NW, BB).astype(jnp.int32)
    qb_t, qc_t, qt_t = _make_kernel(s, b)(ids_t, base_table, context_table)
    return (
        qb_t.transpose(2, 0, 1),
        qc_t.transpose(2, 0, 1),
        qt_t.transpose(2, 0, 1),
    )


# speedup vs baseline: 1.6283x; 1.2110x over previous
import functools

import jax
import jax.numpy as jnp
from jax import lax
from jax.experimental import pallas as pl
from jax.experimental.pallas import tpu as pltpu
from jax.experimental.pallas import tpu_sc as plsc

DIM = 32
EPS = 1e-08
NW = 32
BB = 128
NBUF = 3


def _rsqrt16(x):
    i = lax.bitcast_convert_type(x, jnp.int32)
    y = lax.bitcast_convert_type(
        jnp.int32(0x5F3759DF) - lax.shift_right_logical(i, 1), jnp.float32
    )
    for _ in range(2):
        y = y * (1.5 - 0.5 * x * y * y)
    return jnp.where(x < jnp.float32(EPS * EPS), jnp.float32(1.0 / EPS), y)


def _make_kernel(n_seq, n_batch):
    mesh = plsc.VectorSubcoreMesh(core_axis_name="c", subcore_axis_name="s")

    @functools.partial(
        pl.kernel,
        out_type=(
            jax.ShapeDtypeStruct((n_seq, DIM // 8, NW, 8, BB), jnp.float32),
            jax.ShapeDtypeStruct((n_seq, DIM // 8, NW, 8, BB), jnp.float32),
            jax.ShapeDtypeStruct((n_seq, 2 * DIM // 8, NW, 8, BB), jnp.float32),
        ),
        mesh=mesh,
        scratch_types=[
            pltpu.VMEM((NBUF, BB), jnp.int32),
            pltpu.VMEM((NBUF, BB, DIM), jnp.float32),
            pltpu.VMEM((NBUF, BB, DIM), jnp.float32),
            pltpu.VMEM((NBUF, DIM, BB), jnp.float32),
            pltpu.VMEM((NBUF, DIM, BB), jnp.float32),
            pltpu.VMEM((NBUF, 2 * DIM, BB), jnp.float32),
            pltpu.SemaphoreType.DMA((NBUF,)),
            pltpu.SemaphoreType.DMA((NBUF,)),
        ],
        compiler_params=pltpu.CompilerParams(
            needs_layout_passes=False, use_tc_tiling_on_sc=False
        ),
    )
    def kern(ids_hbm, base_hbm, ctx_hbm, qb_hbm, qc_hbm, qt_hbm,
             idx_v, base_v, ctx_v, tb_v, tc_v, tt_v, gsem, osem):
        wid = lax.axis_index("s") * 2 + lax.axis_index("c")
        col0 = wid * BB
        lanes = lax.iota(jnp.int32, 16)

        def issue_gathers(s, slot):
            pltpu.sync_copy(ids_hbm.at[s, wid], idx_v.at[slot])
            pltpu.async_copy(
                base_hbm.at[idx_v.at[slot]], base_v.at[slot], gsem.at[slot])
            pltpu.async_copy(
                ctx_hbm.at[idx_v.at[slot]], ctx_v.at[slot], gsem.at[slot])

        def wait_gathers(s, slot):
            pltpu.make_async_copy(
                base_hbm.at[idx_v.at[slot]], base_v.at[slot],
                gsem.at[slot]).wait()
            pltpu.make_async_copy(
                ctx_hbm.at[idx_v.at[slot]], ctx_v.at[slot],
                gsem.at[slot]).wait()

        def issue_outputs(s, slot):
            for t0 in range(DIM // 8):
                pltpu.async_copy(
                    tb_v.at[slot, pl.ds(8 * t0, 8)],
                    qb_hbm.at[s, t0, wid], osem.at[slot])
                pltpu.async_copy(
                    tc_v.at[slot, pl.ds(8 * t0, 8)],
                    qc_hbm.at[s, t0, wid], osem.at[slot])
            for t0 in range(2 * DIM // 8):
                pltpu.async_copy(
                    tt_v.at[slot, pl.ds(8 * t0, 8)],
                    qt_hbm.at[s, t0, wid], osem.at[slot])

        def drain_outputs(s, slot):
            for t0 in range(DIM // 8):
                pltpu.make_async_copy(
                    tb_v.at[slot, pl.ds(8 * t0, 8)],
                    qb_hbm.at[s, t0, wid], osem.at[slot]).wait()
                pltpu.make_async_copy(
                    tc_v.at[slot, pl.ds(8 * t0, 8)],
                    qc_hbm.at[s, t0, wid], osem.at[slot]).wait()
            for t0 in range(2 * DIM // 8):
                pltpu.make_async_copy(
                    tt_v.at[slot, pl.ds(8 * t0, 8)],
                    qt_hbm.at[s, t0, wid], osem.at[slot]).wait()

        def compute(slot):
            def group_body(g, _):
                r0 = g * 16
                ridx = r0 + lanes
                acc = [jnp.zeros((16,), jnp.float32) for _ in range(4)]
                for d in range(DIM):
                    didx = (lanes + d) & (DIM - 1)
                    vb = plsc.load_gather(base_v.at[slot], [ridx, didx])
                    plsc.store_scatter(tb_v.at[slot], [didx, ridx], vb)
                    acc[d % 4] = acc[d % 4] + vb * vb
                for d in range(DIM):
                    didx = (lanes + d) & (DIM - 1)
                    vc = plsc.load_gather(ctx_v.at[slot], [ridx, didx])
                    plsc.store_scatter(tc_v.at[slot], [didx, ridx], vc)
                    acc[d % 4] = acc[d % 4] + vc * vc
                inv = _rsqrt16((acc[0] + acc[1]) + (acc[2] + acc[3]))
                for d in range(DIM):
                    tt_v[slot, d, pl.ds(r0, 16)] = (
                        tb_v[slot, d, pl.ds(r0, 16)] * inv)
                    tt_v[slot, DIM + d, pl.ds(r0, 16)] = (
                        tc_v[slot, d, pl.ds(r0, 16)] * inv)
                return 0

            lax.fori_loop(0, BB // 16, group_body, 0)

        issue_gathers(0, 0)

        def tile_body(s, _):
            slot = lax.rem(s, NBUF)
            nslot = lax.rem(s + 1, NBUF)

            @pl.when(s >= 2)
            def _():
                drain_outputs(s - 2, nslot)

            @pl.when(s + 1 < n_seq)
            def _():
                issue_gathers(s + 1, nslot)

            wait_gathers(s, slot)
            compute(slot)
            issue_outputs(s, slot)
            return 0

        lax.fori_loop(0, n_seq, tile_body, 0)
        drain_outputs(n_seq - 2, lax.rem(n_seq - 2, NBUF))
        drain_outputs(n_seq - 1, lax.rem(n_seq - 1, NBUF))

    return kern


def kernel(concept_ids, base_table, context_table):
    b, s = concept_ids.shape
    ids_t = concept_ids.T.reshape(s, NW, BB).astype(jnp.int32)
    qb_t, qc_t, qt_t = _make_kernel(s, b)(ids_t, base_table, context_table)

    def detile(x, d):
        return x.transpose(2, 4, 0, 1, 3).reshape(b, s, d)

    return (detile(qb_t, DIM), detile(qc_t, DIM), detile(qt_t, 2 * DIM))
